# baseline (speedup 1.0000x reference)
import functools

import jax
import jax.numpy as jnp
from jax import lax
from jax.experimental import pallas as pl
from jax.experimental.pallas import tpu as pltpu
from jax.experimental.pallas import tpu_sc as plsc

N_TOKENS = 16384
NGRAMS = 12
VAL_DIM = 32
KEY_DIM = 64
NUM_LM = 32
TOKEN_DIM = 256

NC = 2
NS = 16
NW = NC * NS
TPW = N_TOKENS // NW
CHUNK = 128
N_CHUNKS = TPW // CHUNK
IDX_ROWS_PER_CHUNK = CHUNK * NGRAMS // 128
KEY_BURSTS = TPW // 128


def _sc_body(ngram_tab, ngram_idx, key_tab, key_idx, val_out, key_out,
             nidx_v, rows_v, val_v, kidx_v, krows_v, sem):
    wid = lax.axis_index("s") * NC + lax.axis_index("c")
    base = wid * TPW

    pltpu.sync_copy(key_idx.at[wid], kidx_v)
    kcopies = [
        pltpu.async_copy(key_tab.at[kidx_v.at[i]],
                         krows_v.at[pl.ds(i * 128, 128)], sem)
        for i in range(KEY_BURSTS)
    ]
    for cp in kcopies:
        cp.wait()
    pltpu.sync_copy(krows_v, key_out.at[pl.ds(base, TPW)])

    pltpu.sync_copy(ngram_idx.at[wid], nidx_v)
    for c in range(N_CHUNKS):
        copies = [
            pltpu.async_copy(ngram_tab.at[nidx_v.at[c * IDX_ROWS_PER_CHUNK + r]],
                             rows_v.at[pl.ds(r * 128, 128)], sem)
            for r in range(IDX_ROWS_PER_CHUNK)
        ]
        for cp in copies:
            cp.wait()

        def tok_body(t, carry):
            for h in range(VAL_DIM // 16):
                acc = rows_v[t * NGRAMS, pl.ds(h * 16, 16)]
                for j in range(1, NGRAMS):
                    acc = acc + rows_v[t * NGRAMS + j, pl.ds(h * 16, 16)]
                val_v[t, pl.ds(h * 16, 16)] = acc * (1.0 / NGRAMS)
            return carry

        lax.fori_loop(0, CHUNK, tok_body, 0)
        pltpu.sync_copy(val_v, val_out.at[pl.ds(base + c * CHUNK, CHUNK)])


@jax.jit
def _sc_gather(ngram_tab, ngram_idx_rows, key_tab, key_idx_rows):
    mesh = plsc.VectorSubcoreMesh(core_axis_name="c", subcore_axis_name="s")
    f = pl.kernel(
        _sc_body,
        out_type=[
            jax.ShapeDtypeStruct((N_TOKENS, VAL_DIM), jnp.float32),
            jax.ShapeDtypeStruct((N_TOKENS, KEY_DIM), jnp.float32),
        ],
        mesh=mesh,
        compiler_params=pltpu.CompilerParams(use_tc_tiling_on_sc=False),
        scratch_types=[
            pltpu.VMEM((N_CHUNKS * IDX_ROWS_PER_CHUNK, 128), jnp.int32),
            pltpu.VMEM((CHUNK * NGRAMS, VAL_DIM), jnp.float32),
            pltpu.VMEM((CHUNK, VAL_DIM), jnp.float32),
            pltpu.VMEM((KEY_BURSTS, 128), jnp.int32),
            pltpu.VMEM((TPW, KEY_DIM), jnp.float32),
            pltpu.SemaphoreType.DMA,
        ],
    )
    return f(ngram_tab, ngram_idx_rows, key_tab, key_idx_rows)


TB = 512


def _tc_body(desc_ref, key_ref, val_ref, lmi_ref, dW_ref, db_ref,
             wk_ref, wv_ref, wd_ref, tb_ref, lmt_ref, out_ref):
    descp = jnp.dot(desc_ref[...], dW_ref[...],
                    preferred_element_type=jnp.float32) + db_ref[...]
    acc = jnp.dot(key_ref[...], wk_ref[...], preferred_element_type=jnp.float32)
    acc = acc + jnp.dot(val_ref[...], wv_ref[...],
                        preferred_element_type=jnp.float32)
    acc = acc + jnp.dot(descp, wd_ref[...], preferred_element_type=jnp.float32)
    idx2 = lmi_ref[0]
    oh = (idx2 == lax.broadcasted_iota(jnp.int32, (TB, NUM_LM), 1))
    acc = acc + jnp.dot(oh.astype(jnp.float32), lmt_ref[...],
                        preferred_element_type=jnp.float32)
    out_ref[...] = acc + tb_ref[...]


@jax.jit
def _tc_project(desc_emb, key_e, val_e, lmi3, desc_W, db2, wk, wv, wd, tb2,
                lm_table):
    n = desc_emb.shape[0]
    grid = (n // TB,)
    return pl.pallas_call(
        _tc_body,
        grid=grid,
        in_specs=[
            pl.BlockSpec((TB, desc_emb.shape[1]), lambda i: (i, 0)),
            pl.BlockSpec((TB, KEY_DIM), lambda i: (i, 0)),
            pl.BlockSpec((TB, VAL_DIM), lambda i: (i, 0)),
            pl.BlockSpec((1, TB, 1), lambda i: (i, 0, 0)),
            pl.BlockSpec(desc_W.shape, lambda i: (0, 0)),
            pl.BlockSpec(db2.shape, lambda i: (0, 0)),
            pl.BlockSpec(wk.shape, lambda i: (0, 0)),
            pl.BlockSpec(wv.shape, lambda i: (0, 0)),
            pl.BlockSpec(wd.shape, lambda i: (0, 0)),
            pl.BlockSpec(tb2.shape, lambda i: (0, 0)),
            pl.BlockSpec(lm_table.shape, lambda i: (0, 0)),
        ],
        out_specs=pl.BlockSpec((TB, TOKEN_DIM), lambda i: (i, 0)),
        out_shape=jax.ShapeDtypeStruct((n, TOKEN_DIM), jnp.float32),
    )(desc_emb, key_e, val_e, lmi3, desc_W, db2, wk, wv, wd, tb2, lm_table)


def kernel(key_idx, ngram_idx, landmark_idx, desc_emb, key_table, ngram_table,
           desc_W, desc_b, tag_W, tag_b, lm_table):
    n, g = ngram_idx.shape
    ngidx_rows = ngram_idx.astype(jnp.int32).reshape(NW, n * g // NW // 128, 128)
    kidx_rows = key_idx.astype(jnp.int32).reshape(NW, n // NW // 128, 128)
    val_e, key_e = _sc_gather(ngram_table, ngidx_rows, key_table, kidx_rows)

    lmi3 = landmark_idx.astype(jnp.int32).reshape(n // TB, TB, 1)
    db2 = desc_b.reshape(1, -1)
    tb2 = tag_b.reshape(1, -1)
    wk = tag_W[:KEY_DIM]
    wv = tag_W[KEY_DIM:KEY_DIM + VAL_DIM]
    wd = tag_W[KEY_DIM + VAL_DIM:]
    return _tc_project(desc_emb, key_e, val_e, lmi3, desc_W, db2, wk, wv, wd,
                       tb2, lm_table)
